# Initial kernel scaffold; baseline (speedup 1.0000x reference)
import jax
import jax.numpy as jnp
from jax.experimental import pallas as pl


def kernel(center, W1, g1, be1, W2, cb2, g2, be2, W3, cb3):
    raise NotImplementedError("write your pallas kernel here")


# speedup vs baseline: 13.0843x; 13.0843x over previous
import numpy as np
import jax
import jax.numpy as jnp
from jax.experimental import pallas as pl

_K = 9
_TQ = 256
_EPS = np.float32(1e-10)
_TWO_PI = np.float32(2 * np.pi)
_PI = np.float32(np.pi)
_POS_DEN = np.float32(np.sqrt(3.0) + 1e-6)


def _feat_lists(feat_ref):
    return [feat_ref[0, c] for c in range(10)]


def _br(x):
    return x.astype(jnp.bfloat16).astype(jnp.float32)


def _knn_feat_kernel(xyz_ref, xyzt_ref, w1_ref, feat_ref, acc_ref, *, n_keys, tq):
    b0 = pl.program_id(0)
    j0 = pl.program_id(1)

    kx = xyz_ref[0, :, 0:1]
    ky = xyz_ref[0, :, 1:2]
    kz = xyz_ref[0, :, 2:3]
    qx = xyzt_ref[0, 0:1, :]
    qy = xyzt_ref[0, 1:2, :]
    qz = xyzt_ref[0, 2:3, :]

    q2 = (qx * qx + qy * qy) + qz * qz
    k2 = (kx * kx + ky * ky) + kz * kz
    kxr, kyr, kzr = _br(kx), _br(ky), _br(kz)
    qxr, qyr, qzr = _br(qx), _br(qy), _br(qz)
    dot = (kxr * qxr + kyr * qyr) + kzr * qzr
    dist = (-2.0 * dot + q2) + k2

    iota = jax.lax.broadcasted_iota(jnp.int32, (n_keys, tq), 0)

    cxs, cys, czs = [], [], []
    for _ in range(_K):
        m = jnp.min(dist, axis=0, keepdims=True)
        sel0 = dist == m
        idx = jnp.min(jnp.where(sel0, iota, n_keys), axis=0, keepdims=True)
        sel = iota == idx
        cxs.append(jnp.sum(jnp.where(sel, kx, 0.0), axis=0, keepdims=True))
        cys.append(jnp.sum(jnp.where(sel, ky, 0.0), axis=0, keepdims=True))
        czs.append(jnp.sum(jnp.where(sel, kz, 0.0), axis=0, keepdims=True))
        dist = jnp.where(sel, jnp.inf, dist)

    gx = jnp.concatenate([cxs[s] - qx for s in range(1, _K)], axis=0)
    gy = jnp.concatenate([cys[s] - qy for s in range(1, _K)], axis=0)
    gz = jnp.concatenate([czs[s] - qz for s in range(1, _K)], axis=0)

    xe = jnp.where(jnp.abs(gx) < _EPS, gx + _EPS, gx)
    ye = jnp.where(jnp.abs(gy) < _EPS, gy + _EPS, gy)
    ph = jnp.arctan2(ye, xe) / _TWO_PI + np.float32(0.5)

    rowi = jax.lax.broadcasted_iota(jnp.int32, (8, tq), 0)
    rank = jnp.zeros((8, tq), jnp.int32)
    for j in range(8):
        pj = ph[j : j + 1, :]
        rank = rank + (ph > pj).astype(jnp.int32)
        rank = rank + ((ph == pj) & (rowi > j)).astype(jnp.int32)

    sxs, sys_, szs = [], [], []
    for t in range(8):
        selt = rank == t
        sxs.append(jnp.sum(jnp.where(selt, gx, 0.0), axis=0, keepdims=True))
        sys_.append(jnp.sum(jnp.where(selt, gy, 0.0), axis=0, keepdims=True))
        szs.append(jnp.sum(jnp.where(selt, gz, 0.0), axis=0, keepdims=True))

    sx = jnp.concatenate(sxs, axis=0)
    sy = jnp.concatenate(sys_, axis=0)
    sz = jnp.concatenate(szs, axis=0)
    rx = jnp.concatenate(sxs[1:] + sxs[:1], axis=0)
    ry = jnp.concatenate(sys_[1:] + sys_[:1], axis=0)
    rz = jnp.concatenate(szs[1:] + szs[:1], axis=0)

    nx = sy * rz - sz * ry
    ny = sz * rx - sx * rz
    nz = sx * ry - sy * rx
    nn = jnp.sqrt((nx * nx + ny * ny) + nz * nz)
    safe = jnp.where(nn < 1e-6, 1.0, nn)
    ux = nx / safe
    uy = ny / safe
    uz = nz / safe
    sgn = jnp.where(ux[0:1, :] > 0, 1.0, -1.0)
    ux = ux * sgn
    uy = uy * sgn
    uz = uz * sgn

    cx = (sx + rx) / 3.0
    cy = (sy + ry) / 3.0
    cz = (sz + rz) / 3.0

    rho = jnp.sqrt(((cx * cx + cy * cy) + cz * cz) + _EPS)
    zdr = cz / jnp.maximum(rho, _EPS)
    zdr = jnp.clip(zdr, -1.0 + _EPS, 1.0 - _EPS)
    theta = jnp.arctan2(jnp.sqrt(1.0 - zdr * zdr), zdr) / _PI
    cxe = jnp.where(jnp.abs(cx) < _EPS, cx + _EPS, cx)
    cye = jnp.where(jnp.abs(cy) < _EPS, cy + _EPS, cy)
    phc = jnp.arctan2(cye, cxe) / _TWO_PI + np.float32(0.5)

    nn2 = jnp.sqrt((ux * ux + uy * uy) + uz * uz)
    deg = nn2 < 1e-6
    nmx = jnp.maximum(nn2, 1e-6)
    snx = jnp.where(deg, 1.0, ux / nmx)
    sny = jnp.where(deg, 0.0, uy / nmx)
    snz = jnp.where(deg, 0.0, uz / nmx)
    ccx = jnp.clip(cx, -1e6, 1e6)
    ccy = jnp.clip(cy, -1e6, 1e6)
    ccz = jnp.clip(cz, -1e6, 1e6)
    pos = ((snx * ccx + sny * ccy) + snz * ccz) / _POS_DEN

    feats = [rho, theta, phc, ux, uy, uz, pos, cx, cy, cz]
    for c in range(10):
        feat_ref[0, c] = feats[c]

    @pl.when((b0 == 0) & (j0 == 0))
    def _init():
        acc_ref[...] = jnp.zeros_like(acc_ref)

    w1r = _br(w1_ref[...])
    fr = [_br(f) for f in feats]
    for d in range(10):
        x1d = fr[0] * w1r[d : d + 1, 0:1]
        for c in range(1, 10):
            x1d = x1d + fr[c] * w1r[d : d + 1, c : c + 1]
        acc_ref[d : d + 1, 0:1] += jnp.sum(x1d, axis=(0, 1), keepdims=True)
        acc_ref[d : d + 1, 1:2] += jnp.sum(x1d * x1d, axis=(0, 1), keepdims=True)


def _mm10(wr, vals, bias_ref=None):
    vr = [_br(v) for v in vals]
    out = []
    for d in range(10):
        acc = vr[0] * wr[d : d + 1, 0:1]
        for c in range(1, 10):
            acc = acc + vr[c] * wr[d : d + 1, c : c + 1]
        if bias_ref is not None:
            acc = acc + bias_ref[0:1, d : d + 1]
        out.append(acc)
    return out


def _bn(xs, st_ref):
    return [
        jnp.maximum(x * st_ref[0:1, d : d + 1] + st_ref[1:2, d : d + 1], 0.0)
        for d, x in enumerate(xs)
    ]


def _stats2_kernel(feat_ref, w1_ref, st1_ref, w2_ref, cb2_ref, acc_ref):
    b0 = pl.program_id(0)
    j0 = pl.program_id(1)
    feats = _feat_lists(feat_ref)
    x1 = _mm10(_br(w1_ref[...]), feats)
    h1 = _bn(x1, st1_ref)
    x2 = _mm10(_br(w2_ref[...]), h1, cb2_ref)

    @pl.when((b0 == 0) & (j0 == 0))
    def _init():
        acc_ref[...] = jnp.zeros_like(acc_ref)

    for d in range(10):
        acc_ref[d : d + 1, 0:1] += jnp.sum(x2[d], axis=(0, 1), keepdims=True)
        acc_ref[d : d + 1, 1:2] += jnp.sum(x2[d] * x2[d], axis=(0, 1), keepdims=True)


def _final_kernel(feat_ref, w1_ref, st1_ref, w2_ref, cb2_ref, st2_ref, w3_ref,
                  cb3_ref, out_ref):
    feats = _feat_lists(feat_ref)
    x1 = _mm10(_br(w1_ref[...]), feats)
    h1 = _bn(x1, st1_ref)
    x2 = _mm10(_br(w2_ref[...]), h1, cb2_ref)
    h2 = _bn(x2, st2_ref)
    x3 = _mm10(_br(w3_ref[...]), h2, cb3_ref)
    for d in range(10):
        out_ref[0, d : d + 1, :] = jnp.sum(x3[d], axis=0, keepdims=True)


def kernel(center, W1, g1, be1, W2, cb2, g2, be2, W3, cb3):
    b, n, _ = center.shape
    tq = _TQ
    grid = (b, n // tq)
    xyzt = jnp.transpose(center, (0, 2, 1))

    feat, acc1 = pl.pallas_call(
        lambda *refs: _knn_feat_kernel(*refs, n_keys=n, tq=tq),
        grid=grid,
        in_specs=[
            pl.BlockSpec((1, n, 3), lambda bi, ji: (bi, 0, 0)),
            pl.BlockSpec((1, 3, tq), lambda bi, ji: (bi, 0, ji)),
            pl.BlockSpec((10, 10), lambda bi, ji: (0, 0)),
        ],
        out_specs=[
            pl.BlockSpec((1, 10, 8, tq), lambda bi, ji: (bi, 0, 0, ji)),
            pl.BlockSpec((16, 128), lambda bi, ji: (0, 0)),
        ],
        out_shape=[
            jax.ShapeDtypeStruct((b, 10, 8, n), jnp.float32),
            jax.ShapeDtypeStruct((16, 128), jnp.float32),
        ],
    )(center, xyzt, W1)

    m = np.float32(b * 8 * n)
    sum1 = acc1[:10, 0]
    sq1 = acc1[:10, 1]
    mean1 = sum1 / m
    var1 = sq1 / m - mean1 * mean1
    s1 = g1 / jnp.sqrt(var1 + 1e-5)
    t1 = be1 - mean1 * s1
    st1 = jnp.stack([s1, t1])

    acc2 = pl.pallas_call(
        _stats2_kernel,
        grid=grid,
        in_specs=[
            pl.BlockSpec((1, 10, 8, tq), lambda bi, ji: (bi, 0, 0, ji)),
            pl.BlockSpec((10, 10), lambda bi, ji: (0, 0)),
            pl.BlockSpec((2, 10), lambda bi, ji: (0, 0)),
            pl.BlockSpec((10, 10), lambda bi, ji: (0, 0)),
            pl.BlockSpec((1, 10), lambda bi, ji: (0, 0)),
        ],
        out_specs=pl.BlockSpec((16, 128), lambda bi, ji: (0, 0)),
        out_shape=jax.ShapeDtypeStruct((16, 128), jnp.float32),
    )(feat, W1, st1, W2, cb2[None, :])

    sum2 = acc2[:10, 0]
    sq2 = acc2[:10, 1]
    mean2 = sum2 / m
    var2 = sq2 / m - mean2 * mean2
    s2 = g2 / jnp.sqrt(var2 + 1e-5)
    t2 = be2 - mean2 * s2
    st2 = jnp.stack([s2, t2])

    out = pl.pallas_call(
        _final_kernel,
        grid=grid,
        in_specs=[
            pl.BlockSpec((1, 10, 8, tq), lambda bi, ji: (bi, 0, 0, ji)),
            pl.BlockSpec((10, 10), lambda bi, ji: (0, 0)),
            pl.BlockSpec((2, 10), lambda bi, ji: (0, 0)),
            pl.BlockSpec((10, 10), lambda bi, ji: (0, 0)),
            pl.BlockSpec((1, 10), lambda bi, ji: (0, 0)),
            pl.BlockSpec((2, 10), lambda bi, ji: (0, 0)),
            pl.BlockSpec((10, 10), lambda bi, ji: (0, 0)),
            pl.BlockSpec((1, 10), lambda bi, ji: (0, 0)),
        ],
        out_specs=pl.BlockSpec((1, 10, tq), lambda bi, ji: (bi, 0, ji)),
        out_shape=jax.ShapeDtypeStruct((b, 10, n), jnp.float32),
    )(feat, W1, st1, W2, cb2[None, :], st2, W3, cb3[None, :])

    return out


# speedup vs baseline: 19.9575x; 1.5253x over previous
import numpy as np
import jax
import jax.numpy as jnp
from jax.experimental import pallas as pl

_K = 9
_TQ = 256
_EPS = np.float32(1e-10)
_TWO_PI = np.float32(2 * np.pi)
_PI = np.float32(np.pi)
_POS_DEN = np.float32(np.sqrt(3.0) + 1e-6)


def _feat_lists(feat_ref):
    return [feat_ref[0, c] for c in range(10)]


def _br(x):
    return x.astype(jnp.bfloat16).astype(jnp.float32)


def _split3(row):
    hi = row.astype(jnp.bfloat16)
    r1 = row - hi.astype(jnp.float32)
    mid = r1.astype(jnp.bfloat16)
    lo = (r1 - mid.astype(jnp.float32)).astype(jnp.bfloat16)
    return hi, mid, lo


def _knn_feat_kernel(xyz_ref, xyztf_ref, xyztq_ref, w1_ref, feat_ref, acc_ref,
                     *, n_keys, tq):
    b0 = pl.program_id(0)
    j0 = pl.program_id(1)

    kx = xyz_ref[0, :, 0:1]
    ky = xyz_ref[0, :, 1:2]
    kz = xyz_ref[0, :, 2:3]
    qx = xyztq_ref[0, 0:1, :]
    qy = xyztq_ref[0, 1:2, :]
    qz = xyztq_ref[0, 2:3, :]

    q2 = (qx * qx + qy * qy) + qz * qz
    k2 = (kx * kx + ky * ky) + kz * kz
    kb = xyz_ref[0].astype(jnp.bfloat16)
    qb = xyztq_ref[0].astype(jnp.bfloat16)
    dot = jnp.dot(kb, qb, preferred_element_type=jnp.float32)
    dist = (-2.0 * dot + q2) + k2

    xr = xyztf_ref[0, 0:1, :]
    yr = xyztf_ref[0, 1:2, :]
    zr = xyztf_ref[0, 2:3, :]
    rows = []
    for r in (xr, yr, zr):
        rows.extend(_split3(r))
    rows.append(jnp.ones((1, n_keys), jnp.bfloat16))
    ii = jax.lax.broadcasted_iota(jnp.int32, (1, n_keys), 1)
    ihi = ((ii >> 6) << 6).astype(jnp.float32)
    ilo = (ii & 63).astype(jnp.float32)
    rows.append(ihi.astype(jnp.bfloat16))
    rows.append(ilo.astype(jnp.bfloat16))
    km = jnp.concatenate(rows, axis=0)

    iota = jax.lax.broadcasted_iota(jnp.int32, (n_keys, tq), 0)

    cxs, cys, czs = [], [], []
    for _ in range(_K):
        m = jnp.min(dist, axis=0, keepdims=True)
        sel0 = dist == m
        selb = sel0.astype(jnp.bfloat16)
        g = jnp.dot(km, selb, preferred_element_type=jnp.float32)
        has_tie = jnp.max(g[9:10, :]) > 1.5

        def _slow():
            idx = jnp.min(jnp.where(sel0, iota, n_keys), axis=0, keepdims=True)
            selt = iota == idx
            cx = jnp.sum(jnp.where(selt, kx, 0.0), axis=0, keepdims=True)
            cy = jnp.sum(jnp.where(selt, ky, 0.0), axis=0, keepdims=True)
            cz = jnp.sum(jnp.where(selt, kz, 0.0), axis=0, keepdims=True)
            return cx, cy, cz, idx

        def _fast():
            cx = (g[0:1, :] + g[1:2, :]) + g[2:3, :]
            cy = (g[3:4, :] + g[4:5, :]) + g[5:6, :]
            cz = (g[6:7, :] + g[7:8, :]) + g[8:9, :]
            idx = (g[10:11, :] + g[11:12, :]).astype(jnp.int32)
            return cx, cy, cz, idx

        cx, cy, cz, idx = jax.lax.cond(has_tie, _slow, _fast)
        cxs.append(cx)
        cys.append(cy)
        czs.append(cz)
        dist = jnp.where(iota == idx, jnp.inf, dist)

    gx = jnp.concatenate([cxs[s] - qx for s in range(1, _K)], axis=0)
    gy = jnp.concatenate([cys[s] - qy for s in range(1, _K)], axis=0)
    gz = jnp.concatenate([czs[s] - qz for s in range(1, _K)], axis=0)

    xe = jnp.where(jnp.abs(gx) < _EPS, gx + _EPS, gx)
    ye = jnp.where(jnp.abs(gy) < _EPS, gy + _EPS, gy)
    ph = jnp.arctan2(ye, xe) / _TWO_PI + np.float32(0.5)

    rowi = jax.lax.broadcasted_iota(jnp.int32, (8, tq), 0)
    rank = jnp.zeros((8, tq), jnp.int32)
    for j in range(8):
        pj = ph[j : j + 1, :]
        rank = rank + (ph > pj).astype(jnp.int32)
        rank = rank + ((ph == pj) & (rowi > j)).astype(jnp.int32)

    sxs, sys_, szs = [], [], []
    for t in range(8):
        selt = rank == t
        sxs.append(jnp.sum(jnp.where(selt, gx, 0.0), axis=0, keepdims=True))
        sys_.append(jnp.sum(jnp.where(selt, gy, 0.0), axis=0, keepdims=True))
        szs.append(jnp.sum(jnp.where(selt, gz, 0.0), axis=0, keepdims=True))

    sx = jnp.concatenate(sxs, axis=0)
    sy = jnp.concatenate(sys_, axis=0)
    sz = jnp.concatenate(szs, axis=0)
    rx = jnp.concatenate(sxs[1:] + sxs[:1], axis=0)
    ry = jnp.concatenate(sys_[1:] + sys_[:1], axis=0)
    rz = jnp.concatenate(szs[1:] + szs[:1], axis=0)

    nx = sy * rz - sz * ry
    ny = sz * rx - sx * rz
    nz = sx * ry - sy * rx
    nn = jnp.sqrt((nx * nx + ny * ny) + nz * nz)
    safe = jnp.where(nn < 1e-6, 1.0, nn)
    ux = nx / safe
    uy = ny / safe
    uz = nz / safe
    sgn = jnp.where(ux[0:1, :] > 0, 1.0, -1.0)
    ux = ux * sgn
    uy = uy * sgn
    uz = uz * sgn

    cx = (sx + rx) / 3.0
    cy = (sy + ry) / 3.0
    cz = (sz + rz) / 3.0

    rho = jnp.sqrt(((cx * cx + cy * cy) + cz * cz) + _EPS)
    zdr = cz / jnp.maximum(rho, _EPS)
    zdr = jnp.clip(zdr, -1.0 + _EPS, 1.0 - _EPS)
    theta = jnp.arctan2(jnp.sqrt(1.0 - zdr * zdr), zdr) / _PI
    cxe = jnp.where(jnp.abs(cx) < _EPS, cx + _EPS, cx)
    cye = jnp.where(jnp.abs(cy) < _EPS, cy + _EPS, cy)
    phc = jnp.arctan2(cye, cxe) / _TWO_PI + np.float32(0.5)

    nn2 = jnp.sqrt((ux * ux + uy * uy) + uz * uz)
    deg = nn2 < 1e-6
    nmx = jnp.maximum(nn2, 1e-6)
    snx = jnp.where(deg, 1.0, ux / nmx)
    sny = jnp.where(deg, 0.0, uy / nmx)
    snz = jnp.where(deg, 0.0, uz / nmx)
    ccx = jnp.clip(cx, -1e6, 1e6)
    ccy = jnp.clip(cy, -1e6, 1e6)
    ccz = jnp.clip(cz, -1e6, 1e6)
    pos = ((snx * ccx + sny * ccy) + snz * ccz) / _POS_DEN

    feats = [rho, theta, phc, ux, uy, uz, pos, cx, cy, cz]
    for c in range(10):
        feat_ref[0, c] = feats[c]

    @pl.when((b0 == 0) & (j0 == 0))
    def _init():
        acc_ref[...] = jnp.zeros_like(acc_ref)

    w1r = _br(w1_ref[...])
    fr = [_br(f) for f in feats]
    for d in range(10):
        x1d = fr[0] * w1r[d : d + 1, 0:1]
        for c in range(1, 10):
            x1d = x1d + fr[c] * w1r[d : d + 1, c : c + 1]
        acc_ref[d : d + 1, 0:1] += jnp.sum(x1d, axis=(0, 1), keepdims=True)
        acc_ref[d : d + 1, 1:2] += jnp.sum(x1d * x1d, axis=(0, 1), keepdims=True)


def _mm10(wr, vals, bias_ref=None):
    vr = [_br(v) for v in vals]
    out = []
    for d in range(10):
        acc = vr[0] * wr[d : d + 1, 0:1]
        for c in range(1, 10):
            acc = acc + vr[c] * wr[d : d + 1, c : c + 1]
        if bias_ref is not None:
            acc = acc + bias_ref[0:1, d : d + 1]
        out.append(acc)
    return out


def _bn(xs, st_ref):
    return [
        jnp.maximum(x * st_ref[0:1, d : d + 1] + st_ref[1:2, d : d + 1], 0.0)
        for d, x in enumerate(xs)
    ]


def _stats2_kernel(feat_ref, w1_ref, st1_ref, w2_ref, cb2_ref, acc_ref):
    b0 = pl.program_id(0)
    j0 = pl.program_id(1)
    feats = _feat_lists(feat_ref)
    x1 = _mm10(_br(w1_ref[...]), feats)
    h1 = _bn(x1, st1_ref)
    x2 = _mm10(_br(w2_ref[...]), h1, cb2_ref)

    @pl.when((b0 == 0) & (j0 == 0))
    def _init():
        acc_ref[...] = jnp.zeros_like(acc_ref)

    for d in range(10):
        acc_ref[d : d + 1, 0:1] += jnp.sum(x2[d], axis=(0, 1), keepdims=True)
        acc_ref[d : d + 1, 1:2] += jnp.sum(x2[d] * x2[d], axis=(0, 1), keepdims=True)


def _final_kernel(feat_ref, w1_ref, st1_ref, w2_ref, cb2_ref, st2_ref, w3_ref,
                  cb3_ref, out_ref):
    feats = _feat_lists(feat_ref)
    x1 = _mm10(_br(w1_ref[...]), feats)
    h1 = _bn(x1, st1_ref)
    x2 = _mm10(_br(w2_ref[...]), h1, cb2_ref)
    h2 = _bn(x2, st2_ref)
    x3 = _mm10(_br(w3_ref[...]), h2, cb3_ref)
    for d in range(10):
        out_ref[0, d : d + 1, :] = jnp.sum(x3[d], axis=0, keepdims=True)


def kernel(center, W1, g1, be1, W2, cb2, g2, be2, W3, cb3):
    b, n, _ = center.shape
    tq = _TQ
    grid = (b, n // tq)
    xyzt = jnp.transpose(center, (0, 2, 1))

    feat, acc1 = pl.pallas_call(
        lambda *refs: _knn_feat_kernel(*refs, n_keys=n, tq=tq),
        grid=grid,
        in_specs=[
            pl.BlockSpec((1, n, 3), lambda bi, ji: (bi, 0, 0)),
            pl.BlockSpec((1, 3, n), lambda bi, ji: (bi, 0, 0)),
            pl.BlockSpec((1, 3, tq), lambda bi, ji: (bi, 0, ji)),
            pl.BlockSpec((10, 10), lambda bi, ji: (0, 0)),
        ],
        out_specs=[
            pl.BlockSpec((1, 10, 8, tq), lambda bi, ji: (bi, 0, 0, ji)),
            pl.BlockSpec((16, 128), lambda bi, ji: (0, 0)),
        ],
        out_shape=[
            jax.ShapeDtypeStruct((b, 10, 8, n), jnp.float32),
            jax.ShapeDtypeStruct((16, 128), jnp.float32),
        ],
    )(center, xyzt, xyzt, W1)

    m = np.float32(b * 8 * n)
    sum1 = acc1[:10, 0]
    sq1 = acc1[:10, 1]
    mean1 = sum1 / m
    var1 = sq1 / m - mean1 * mean1
    s1 = g1 / jnp.sqrt(var1 + 1e-5)
    t1 = be1 - mean1 * s1
    st1 = jnp.stack([s1, t1])

    acc2 = pl.pallas_call(
        _stats2_kernel,
        grid=grid,
        in_specs=[
            pl.BlockSpec((1, 10, 8, tq), lambda bi, ji: (bi, 0, 0, ji)),
            pl.BlockSpec((10, 10), lambda bi, ji: (0, 0)),
            pl.BlockSpec((2, 10), lambda bi, ji: (0, 0)),
            pl.BlockSpec((10, 10), lambda bi, ji: (0, 0)),
            pl.BlockSpec((1, 10), lambda bi, ji: (0, 0)),
        ],
        out_specs=pl.BlockSpec((16, 128), lambda bi, ji: (0, 0)),
        out_shape=jax.ShapeDtypeStruct((16, 128), jnp.float32),
    )(feat, W1, st1, W2, cb2[None, :])

    sum2 = acc2[:10, 0]
    sq2 = acc2[:10, 1]
    mean2 = sum2 / m
    var2 = sq2 / m - mean2 * mean2
    s2 = g2 / jnp.sqrt(var2 + 1e-5)
    t2 = be2 - mean2 * s2
    st2 = jnp.stack([s2, t2])

    out = pl.pallas_call(
        _final_kernel,
        grid=grid,
        in_specs=[
            pl.BlockSpec((1, 10, 8, tq), lambda bi, ji: (bi, 0, 0, ji)),
            pl.BlockSpec((10, 10), lambda bi, ji: (0, 0)),
            pl.BlockSpec((2, 10), lambda bi, ji: (0, 0)),
            pl.BlockSpec((10, 10), lambda bi, ji: (0, 0)),
            pl.BlockSpec((1, 10), lambda bi, ji: (0, 0)),
            pl.BlockSpec((2, 10), lambda bi, ji: (0, 0)),
            pl.BlockSpec((10, 10), lambda bi, ji: (0, 0)),
            pl.BlockSpec((1, 10), lambda bi, ji: (0, 0)),
        ],
        out_specs=pl.BlockSpec((1, 10, tq), lambda bi, ji: (bi, 0, ji)),
        out_shape=jax.ShapeDtypeStruct((b, 10, n), jnp.float32),
    )(feat, W1, st1, W2, cb2[None, :], st2, W3, cb3[None, :])

    return out
